# baseline (speedup 1.0000x reference)
import functools

import jax
import jax.numpy as jnp
from jax import lax
from jax.experimental import pallas as pl
from jax.experimental.pallas import tpu as pltpu
from jax.experimental.pallas import tpu_sc as plsc

N = 10000
E = 320000
ATOM_F = 128
DIP_F = 64

_NS = 16
_B = 128
_NCHUNKS = E // _B
_C = _NCHUNKS // _NS
_XTRA = _NCHUNKS - _C * _NS
_ROWS_PT = N // _NS


def _mlp_body(x_ref, w1_ref, b1_ref, w2_ref, b2_ref, v_ref, out_ref, vt_ref):
    h = jnp.dot(x_ref[...], w1_ref[...], preferred_element_type=jnp.float32)
    h = h + b1_ref[...]
    h = h * jax.nn.sigmoid(h)
    q = jnp.dot(h, w2_ref[...], preferred_element_type=jnp.float32)
    q = q + b2_ref[...]
    q = q * jax.nn.sigmoid(q)
    out_ref[0] = q[:, :32]
    out_ref[1] = q[:, 32:]
    vt_ref[...] = v_ref[...].T


def _mlp(x, W1, b1, W2, b2, vij):
    R = 1000
    EB = E // (N // R)
    grid = (N // R,)
    return pl.pallas_call(
        _mlp_body,
        grid=grid,
        in_specs=[
            pl.BlockSpec((R, ATOM_F), lambda i: (i, 0)),
            pl.BlockSpec((ATOM_F, ATOM_F), lambda i: (0, 0)),
            pl.BlockSpec((1, ATOM_F), lambda i: (0, 0)),
            pl.BlockSpec((ATOM_F, DIP_F), lambda i: (0, 0)),
            pl.BlockSpec((1, DIP_F), lambda i: (0, 0)),
            pl.BlockSpec((EB, 3), lambda i: (i, 0)),
        ],
        out_specs=[pl.BlockSpec((2, R, 32), lambda i: (0, i, 0)),
                   pl.BlockSpec((3, EB), lambda i: (0, i))],
        out_shape=[jax.ShapeDtypeStruct((2, N, 32), jnp.float32),
                   jax.ShapeDtypeStruct((3, E), jnp.float32)],
    )(x, W1, b1.reshape(1, ATOM_F), W2, b2.reshape(1, DIP_F), vij)


_sc_mesh = plsc.VectorSubcoreMesh(core_axis_name="c", subcore_axis_name="s")


@functools.partial(
    pl.kernel,
    out_type=jax.ShapeDtypeStruct((2, N, 3, 32), jnp.float32),
    mesh=_sc_mesh,
    scratch_types=[
        pltpu.VMEM((4, _B), jnp.int32),
        pltpu.VMEM((4, _B), jnp.int32),
        pltpu.VMEM((4, 3, _B), jnp.float32),
        pltpu.VMEM((4, _B, 32), jnp.float32),
        pltpu.VMEM((2, _B, 3, 32), jnp.float32),
        pltpu.VMEM((2, _B), jnp.int32),
        pltpu.VMEM_SHARED((N, 3, 32), jnp.float32),
        pltpu.SemaphoreType.DMA,
        pltpu.SemaphoreType.DMA,
        pltpu.SemaphoreType.DMA,
    ],
    compiler_params=pltpu.CompilerParams(use_tc_tiling_on_sc=False),
)
def _sc_edge(q0_hbm, q1_hbm, ei_hbm, vij_hbm, out_hbm,
             sidx, didx, vv, qrows, msg, dbuf, acc, semi, semg, sems):
    ci = lax.axis_index("c")
    si = lax.axis_index("s")
    row0 = si * _ROWS_PT
    ch0 = si * _C + jnp.minimum(si, _XTRA)

    zero16 = jnp.zeros((16,), jnp.float32)

    @plsc.parallel_loop(0, 125)
    def _zrow(r):
        for c in range(3):
            msg[0, r, c, pl.ds(0, 16)] = zero16
            msg[0, r, c, pl.ds(16, 16)] = zero16

    for i in range(5):
        pltpu.sync_copy(msg.at[0, pl.ds(0, 125)],
                        acc.at[pl.ds(row0 + i * 125, 125)])
    plsc.subcore_barrier()

    def fire_loads(kk, slot):
        ch = ch0 + kk
        pltpu.async_copy(ei_hbm.at[0, ch], sidx.at[slot], semi)
        pltpu.async_copy(ei_hbm.at[1, ch], didx.at[slot], semi)
        pltpu.async_copy(vij_hbm.at[:, pl.ds(ch * _B, _B)], vv.at[slot], semi)

    def wait_loads(slot):
        pltpu.make_async_copy(ei_hbm.at[0, 0], sidx.at[slot], semi).wait()
        pltpu.make_async_copy(ei_hbm.at[1, 0], didx.at[slot], semi).wait()
        pltpu.make_async_copy(vij_hbm.at[:, pl.ds(0, _B)], vv.at[slot],
                              semi).wait()

    def fire_gather(slot):
        idx = sidx.at[slot]
        dst = qrows.at[slot]

        @pl.when(ci == 0)
        def _():
            pltpu.async_copy(q0_hbm.at[idx], dst, semg)

        @pl.when(ci == 1)
        def _():
            pltpu.async_copy(q1_hbm.at[idx], dst, semg)

    def wait_gather(slot):
        pltpu.make_async_copy(q0_hbm.at[pl.ds(0, _B)], qrows.at[slot],
                              semg).wait()

    def wait_scatter(m):
        pltpu.make_async_copy(out_hbm.at[0, pl.ds(0, _B)], msg.at[m],
                              sems).wait()

    def compute(slot, m):
        @plsc.parallel_loop(0, _B // 16, unroll=2)
        def _group(g):
            gsl = pl.ds(g * 16, 16)
            v0g = vv[slot, 0, gsl]
            v1g = vv[slot, 1, gsl]
            v2g = vv[slot, 2, gsl]
            for jj in range(16):
                j = g * 16 + jj
                q0 = qrows[slot, j, pl.ds(0, 16)]
                q1 = qrows[slot, j, pl.ds(16, 16)]
                v0 = v0g[jj]
                v1 = v1g[jj]
                v2 = v2g[jj]
                msg[m, j, 0, pl.ds(0, 16)] = q0 * v0
                msg[m, j, 0, pl.ds(16, 16)] = q1 * v0
                msg[m, j, 1, pl.ds(0, 16)] = q0 * v1
                msg[m, j, 1, pl.ds(16, 16)] = q1 * v1
                msg[m, j, 2, pl.ds(0, 16)] = q0 * v2
                msg[m, j, 2, pl.ds(16, 16)] = q1 * v2

    def fire_scatter(slot, m):
        for k in range(_B // 16):
            ksl = pl.ds(k * 16, 16)
            dbuf[m, ksl] = didx[slot, ksl]
        pltpu.async_copy(msg.at[m], acc.at[dbuf.at[m]], sems, add=True)

    fire_loads(0, 0)
    fire_loads(1, 1)
    fire_loads(2, 2)
    wait_loads(0)
    fire_gather(0)
    wait_loads(1)
    fire_gather(1)

    def quad_body(k4, _):
        for b in range(4):
            kk = k4 + b
            m = b % 2

            if b >= 2:
                wait_scatter(m)
            else:
                @pl.when(k4 > 0)
                def _():
                    wait_scatter(m)

            if b == 0:
                fire_loads(kk + 3, (b + 3) % 4)
            else:
                @pl.when(k4 < _C - 4)
                def _():
                    fire_loads(kk + 3, (b + 3) % 4)

            if b < 2:
                wait_loads((b + 2) % 4)
                fire_gather((b + 2) % 4)
            else:
                @pl.when(k4 < _C - 4)
                def _():
                    wait_loads((b + 2) % 4)
                    fire_gather((b + 2) % 4)

            wait_gather(b)
            compute(b, m)
            fire_scatter(b, m)
        return 0

    lax.fori_loop(0, _C // 4, lambda i, c: quad_body(i * 4, c), 0)
    wait_scatter(0)
    wait_scatter(1)

    @pl.when(si < _XTRA)
    def _():
        fire_loads(_C, 0)
        wait_loads(0)
        fire_gather(0)
        wait_gather(0)
        compute(0, 0)
        fire_scatter(0, 0)
        wait_scatter(0)

    plsc.subcore_barrier()

    pltpu.sync_copy(acc.at[pl.ds(row0, _ROWS_PT)],
                    out_hbm.at[ci, pl.ds(row0, _ROWS_PT)])


@jax.jit
def kernel(x, rij, vij, edge_index, W1, b1, W2, b2):
    del rij
    ei = edge_index.astype(jnp.int32).reshape(2, _NCHUNKS, _B)
    qh, vijT = _mlp(x, W1, b1, W2, b2, vij)
    out = _sc_edge(qh[0], qh[1], ei, vijT)
    return out.transpose(1, 0, 3, 2).reshape(N, DIP_F, 3)


# speedup vs baseline: 1.5286x; 1.5286x over previous
import functools

import jax
import jax.numpy as jnp
from jax import lax
from jax.experimental import pallas as pl
from jax.experimental.pallas import tpu as pltpu
from jax.experimental.pallas import tpu_sc as plsc

N = 10000
E = 320000
ATOM_F = 128
DIP_F = 64

_NS = 16
_B = 128
_NCHUNKS = E // _B
_C = _NCHUNKS // _NS
_XTRA = _NCHUNKS - _C * _NS
_ROWS_PT = N // _NS


def _mlp_body(x_ref, w1_ref, b1_ref, w2_ref, b2_ref, out_ref):
    h = jnp.dot(x_ref[...], w1_ref[...], preferred_element_type=jnp.float32)
    h = h + b1_ref[...]
    h = h * jax.nn.sigmoid(h)
    q = jnp.dot(h, w2_ref[...], preferred_element_type=jnp.float32)
    q = q + b2_ref[...]
    q = q * jax.nn.sigmoid(q)
    out_ref[0] = q[:, :32]
    out_ref[1] = q[:, 32:]


def _mlp(x, W1, b1, W2, b2):
    R = 1000
    grid = (N // R,)
    return pl.pallas_call(
        _mlp_body,
        grid=grid,
        in_specs=[
            pl.BlockSpec((R, ATOM_F), lambda i: (i, 0)),
            pl.BlockSpec((ATOM_F, ATOM_F), lambda i: (0, 0)),
            pl.BlockSpec((1, ATOM_F), lambda i: (0, 0)),
            pl.BlockSpec((ATOM_F, DIP_F), lambda i: (0, 0)),
            pl.BlockSpec((1, DIP_F), lambda i: (0, 0)),
        ],
        out_specs=pl.BlockSpec((2, R, 32), lambda i: (0, i, 0)),
        out_shape=jax.ShapeDtypeStruct((2, N, 32), jnp.float32),
    )(x, W1, b1.reshape(1, ATOM_F), W2, b2.reshape(1, DIP_F))


_sc_mesh = plsc.VectorSubcoreMesh(core_axis_name="c", subcore_axis_name="s")


@functools.partial(
    pl.kernel,
    out_type=jax.ShapeDtypeStruct((2, N, 3, 32), jnp.float32),
    mesh=_sc_mesh,
    scratch_types=[
        pltpu.VMEM((4, _B), jnp.int32),
        pltpu.VMEM((4, _B), jnp.int32),
        pltpu.VMEM((4, 3, _B), jnp.float32),
        pltpu.VMEM((4, _B, 32), jnp.float32),
        pltpu.VMEM((2, _B, 3, 32), jnp.float32),
        pltpu.VMEM((2, _B), jnp.int32),
        pltpu.VMEM_SHARED((N, 3, 32), jnp.float32),
        pltpu.SemaphoreType.DMA,
        pltpu.SemaphoreType.DMA,
        pltpu.SemaphoreType.DMA,
    ],
    compiler_params=pltpu.CompilerParams(use_tc_tiling_on_sc=False),
)
def _sc_edge(q0_hbm, q1_hbm, ei_hbm, vij_hbm, out_hbm,
             sidx, didx, vv, qrows, msg, dbuf, acc, semi, semg, sems):
    ci = lax.axis_index("c")
    si = lax.axis_index("s")
    row0 = si * _ROWS_PT
    ch0 = si * _C + jnp.minimum(si, _XTRA)

    zero16 = jnp.zeros((16,), jnp.float32)

    @plsc.parallel_loop(0, 125)
    def _zrow(r):
        for c in range(3):
            msg[0, r, c, pl.ds(0, 16)] = zero16
            msg[0, r, c, pl.ds(16, 16)] = zero16

    for i in range(5):
        pltpu.sync_copy(msg.at[0, pl.ds(0, 125)],
                        acc.at[pl.ds(row0 + i * 125, 125)])
    plsc.subcore_barrier()

    def fire_loads(kk, slot):
        ch = ch0 + kk
        pltpu.async_copy(ei_hbm.at[0, ch], sidx.at[slot], semi)
        pltpu.async_copy(ei_hbm.at[1, ch], didx.at[slot], semi)
        pltpu.async_copy(vij_hbm.at[:, pl.ds(ch * _B, _B)], vv.at[slot], semi)

    def wait_loads(slot):
        pltpu.make_async_copy(ei_hbm.at[0, 0], sidx.at[slot], semi).wait()
        pltpu.make_async_copy(ei_hbm.at[1, 0], didx.at[slot], semi).wait()
        pltpu.make_async_copy(vij_hbm.at[:, pl.ds(0, _B)], vv.at[slot],
                              semi).wait()

    def fire_gather(slot):
        idx = sidx.at[slot]
        dst = qrows.at[slot]

        @pl.when(ci == 0)
        def _():
            pltpu.async_copy(q0_hbm.at[idx], dst, semg)

        @pl.when(ci == 1)
        def _():
            pltpu.async_copy(q1_hbm.at[idx], dst, semg)

    def wait_gather(slot):
        pltpu.make_async_copy(q0_hbm.at[pl.ds(0, _B)], qrows.at[slot],
                              semg).wait()

    def wait_scatter(m):
        pltpu.make_async_copy(out_hbm.at[0, pl.ds(0, _B)], msg.at[m],
                              sems).wait()

    def compute(slot, m):
        @plsc.parallel_loop(0, _B // 16, unroll=2)
        def _group(g):
            gsl = pl.ds(g * 16, 16)
            v0g = vv[slot, 0, gsl]
            v1g = vv[slot, 1, gsl]
            v2g = vv[slot, 2, gsl]
            for jj in range(16):
                j = g * 16 + jj
                q0 = qrows[slot, j, pl.ds(0, 16)]
                q1 = qrows[slot, j, pl.ds(16, 16)]
                v0 = v0g[jj]
                v1 = v1g[jj]
                v2 = v2g[jj]
                msg[m, j, 0, pl.ds(0, 16)] = q0 * v0
                msg[m, j, 0, pl.ds(16, 16)] = q1 * v0
                msg[m, j, 1, pl.ds(0, 16)] = q0 * v1
                msg[m, j, 1, pl.ds(16, 16)] = q1 * v1
                msg[m, j, 2, pl.ds(0, 16)] = q0 * v2
                msg[m, j, 2, pl.ds(16, 16)] = q1 * v2

    def fire_scatter(slot, m):
        for k in range(_B // 16):
            ksl = pl.ds(k * 16, 16)
            dbuf[m, ksl] = didx[slot, ksl]
        pltpu.async_copy(msg.at[m], acc.at[dbuf.at[m]], sems, add=True)

    fire_loads(0, 0)
    fire_loads(1, 1)
    fire_loads(2, 2)
    wait_loads(0)
    fire_gather(0)
    wait_loads(1)
    fire_gather(1)

    def quad_body(k4, _):
        for b in range(4):
            kk = k4 + b
            m = b % 2

            if b >= 2:
                wait_scatter(m)
            else:
                @pl.when(k4 > 0)
                def _():
                    wait_scatter(m)

            if b == 0:
                fire_loads(kk + 3, (b + 3) % 4)
            else:
                @pl.when(k4 < _C - 4)
                def _():
                    fire_loads(kk + 3, (b + 3) % 4)

            if b < 2:
                wait_loads((b + 2) % 4)
                fire_gather((b + 2) % 4)
            else:
                @pl.when(k4 < _C - 4)
                def _():
                    wait_loads((b + 2) % 4)
                    fire_gather((b + 2) % 4)

            wait_gather(b)
            compute(b, m)
            fire_scatter(b, m)
        return 0

    lax.fori_loop(0, _C // 4, lambda i, c: quad_body(i * 4, c), 0)
    wait_scatter(0)
    wait_scatter(1)

    @pl.when(si < _XTRA)
    def _():
        fire_loads(_C, 0)
        wait_loads(0)
        fire_gather(0)
        wait_gather(0)
        compute(0, 0)
        fire_scatter(0, 0)
        wait_scatter(0)

    plsc.subcore_barrier()

    pltpu.sync_copy(acc.at[pl.ds(row0, _ROWS_PT)],
                    out_hbm.at[ci, pl.ds(row0, _ROWS_PT)])


@jax.jit
def kernel(x, rij, vij, edge_index, W1, b1, W2, b2):
    del rij
    ei = edge_index.astype(jnp.int32).reshape(2, _NCHUNKS, _B)
    vijT = vij.T
    qh = _mlp(x, W1, b1, W2, b2)
    out = _sc_edge(qh[0], qh[1], ei, vijT)
    return out.transpose(1, 0, 3, 2).reshape(N, DIP_F, 3)
